# baseline (speedup 1.0000x reference)
import functools

import jax
import jax.numpy as jnp
from jax.experimental import pallas as pl

_NB = 4
_K = 1024
_D = 128
_ROWS = 8 * 576


def _vq_body(z_ref, cb_ref, codes_ref, inds_ref, comm_ref):
    i = pl.program_id(0)
    z = z_ref[...]
    cb = cb_ref[0]
    c2h = 0.5 * jnp.sum(cb * cb, axis=1)
    dots = jnp.dot(z, cb.T, preferred_element_type=jnp.float32)
    z2 = jnp.sum(z * z, axis=1)
    v = dots - c2h[None, :]
    vmax = jnp.max(v, axis=1)
    m = z2 - 2.0 * vmax
    lane_row = jax.lax.broadcasted_iota(jnp.int32, (1, _K), 1)
    lane_f = lane_row.astype(jnp.float32)
    idx_f = jnp.min(jnp.where(v == vmax[:, None], lane_f,
                              jnp.float32(_K)), axis=1)
    idx = idx_f.astype(jnp.int32)
    onehot = (lane_row == idx[:, None]).astype(jnp.bfloat16)
    q = jnp.dot(onehot, cb.astype(jnp.bfloat16),
                preferred_element_type=jnp.float32)
    codes_ref[...] = q

    col = jax.lax.broadcasted_iota(jnp.int32, (1, _NB), 1)
    @pl.when(i == 0)
    def _init_inds():
        inds_ref[...] = jnp.zeros((_ROWS, _NB), jnp.int32)
    inds_ref[...] = jnp.where(col == i, idx[:, None], inds_ref[...])

    s = jnp.sum(m) / jnp.float32(_ROWS * _D)
    lane4 = jax.lax.broadcasted_iota(jnp.int32, (1, _NB), 1)
    @pl.when(i == 0)
    def _init_comm():
        comm_ref[...] = jnp.zeros((1, _NB), jnp.float32)
    comm_ref[...] = jnp.where(lane4 == i, s, comm_ref[...])


@functools.partial(jax.jit)
def kernel(x, codebooks):
    b, n, D = x.shape
    xr = x.reshape(b * n, D)
    codes, inds, comm = pl.pallas_call(
        _vq_body,
        grid=(_NB,),
        in_specs=[
            pl.BlockSpec((_ROWS, _D), lambda i: (0, i)),
            pl.BlockSpec((1, _K, _D), lambda i: (i, 0, 0)),
        ],
        out_specs=[
            pl.BlockSpec((_ROWS, _D), lambda i: (0, i)),
            pl.BlockSpec((_ROWS, _NB), lambda i: (0, 0)),
            pl.BlockSpec((1, _NB), lambda i: (0, 0)),
        ],
        out_shape=[
            jax.ShapeDtypeStruct((_ROWS, _NB * _D), jnp.float32),
            jax.ShapeDtypeStruct((_ROWS, _NB), jnp.int32),
            jax.ShapeDtypeStruct((1, _NB), jnp.float32),
        ],
    )(xr, codebooks)
    return (codes.reshape(b, n, D), inds.reshape(b, n, _NB),
            comm.reshape(_NB))


# speedup vs baseline: 1.0592x; 1.0592x over previous
import functools

import jax
import jax.numpy as jnp
from jax.experimental import pallas as pl

_NB = 4
_K = 1024
_D = 128
_ROWS = 8 * 576


def _vq_body(z_ref, cb_ref, codes_ref, inds_ref, comm_ref):
    i = pl.program_id(0)
    z = z_ref[...]
    cb = cb_ref[0]
    c2h = 0.5 * jnp.sum(cb * cb, axis=1)
    one_col = (jax.lax.broadcasted_iota(jnp.int32, (1, 8), 1)
               == 0).astype(jnp.float32)
    z_aug = jnp.concatenate(
        [z, jnp.broadcast_to(one_col, (_ROWS, 8))], axis=1)
    cb_aug = jnp.concatenate(
        [cb, -c2h[:, None], jnp.zeros((_K, 7), jnp.float32)],
        axis=1)
    v = jax.lax.dot_general(z_aug, cb_aug, (((1,), (1,)), ((), ())),
                            preferred_element_type=jnp.float32)
    z2 = jnp.sum(z * z, axis=1)
    vmax = jnp.max(v, axis=1)
    m = z2 - 2.0 * vmax
    lane_row = jax.lax.broadcasted_iota(jnp.int32, (1, _K), 1)
    lane_f = lane_row.astype(jnp.float32)
    idx_f = jnp.min(jnp.where(v == vmax[:, None], lane_f,
                              jnp.float32(_K)), axis=1)
    idx = idx_f.astype(jnp.int32)
    onehot = (lane_row == idx[:, None]).astype(jnp.bfloat16)
    q = jnp.dot(onehot, cb.astype(jnp.bfloat16),
                preferred_element_type=jnp.float32)
    codes_ref[...] = q

    col = jax.lax.broadcasted_iota(jnp.int32, (1, _NB), 1)
    @pl.when(i == 0)
    def _init_inds():
        inds_ref[...] = jnp.zeros((_ROWS, _NB), jnp.int32)
    inds_ref[...] = jnp.where(col == i, idx[:, None], inds_ref[...])

    s = jnp.sum(m) / jnp.float32(_ROWS * _D)
    lane4 = jax.lax.broadcasted_iota(jnp.int32, (1, _NB), 1)
    @pl.when(i == 0)
    def _init_comm():
        comm_ref[...] = jnp.zeros((1, _NB), jnp.float32)
    comm_ref[...] = jnp.where(lane4 == i, s, comm_ref[...])


@functools.partial(jax.jit)
def kernel(x, codebooks):
    b, n, D = x.shape
    xr = x.reshape(b * n, D)
    codes, inds, comm = pl.pallas_call(
        _vq_body,
        grid=(_NB,),
        in_specs=[
            pl.BlockSpec((_ROWS, _D), lambda i: (0, i)),
            pl.BlockSpec((1, _K, _D), lambda i: (i, 0, 0)),
        ],
        out_specs=[
            pl.BlockSpec((_ROWS, _D), lambda i: (0, i)),
            pl.BlockSpec((_ROWS, _NB), lambda i: (0, 0)),
            pl.BlockSpec((1, _NB), lambda i: (0, 0)),
        ],
        out_shape=[
            jax.ShapeDtypeStruct((_ROWS, _NB * _D), jnp.float32),
            jax.ShapeDtypeStruct((_ROWS, _NB), jnp.int32),
            jax.ShapeDtypeStruct((1, _NB), jnp.float32),
        ],
    )(xr, codebooks)
    return (codes.reshape(b, n, D), inds.reshape(b, n, _NB),
            comm.reshape(_NB))
